# baseline (speedup 1.0000x reference)
import jax
import jax.numpy as jnp
from jax.experimental import pallas as pl
from jax.experimental.pallas import tpu as pltpu

_K = 8
_BLK = 2048
_DK = 2048


def _router_block(x_ref, w0_ref, w1_ref, fw_ref, idx_ref, acc_ref):
    blk, e = fw_ref.shape
    kk = pl.program_id(1)

    @pl.when(kk == 0)
    def _():
        acc_ref[...] = jax.lax.dot_general(
            x_ref[...].astype(jnp.bfloat16), w0_ref[...].astype(jnp.bfloat16),
            (((1,), (1,)), ((), ())),
            preferred_element_type=jnp.float32,
            precision=jax.lax.Precision.DEFAULT)

    @pl.when(kk == 1)
    def _():
        logits = acc_ref[...] + jax.lax.dot_general(
            x_ref[...].astype(jnp.bfloat16), w1_ref[...].astype(jnp.bfloat16),
            (((1,), (1,)), ((), ())),
            preferred_element_type=jnp.float32,
            precision=jax.lax.Precision.DEFAULT)
        iota = jax.lax.broadcasted_iota(
            jnp.int32, (blk, e), 1).astype(jnp.float32)
        cur = logits
        idx_cols = []
        m0 = None
        for k in range(_K):
            m = jnp.max(cur, axis=1, keepdims=True)
            if k == 0:
                m0 = m
            amax = jnp.min(jnp.where(cur == m, iota, float(e)), axis=1,
                           keepdims=True)
            idx_cols.append(amax)
            cur = jnp.where(iota == amax, -jnp.inf, cur)
        sel = cur == -jnp.inf
        ex = jnp.where(sel, jnp.exp(logits - m0), 0.0)
        z = jnp.sum(ex, axis=1, keepdims=True)
        fw_ref[...] = ex / z
        idx_ref[...] = jnp.concatenate(idx_cols, axis=1).astype(jnp.int32)


def kernel(x, W):
    b, d = x.shape
    e = W.shape[0]
    w0 = W[:, :_DK]
    w1 = W[:, _DK:]
    fw, idx = pl.pallas_call(
        _router_block,
        grid=(b // _BLK, 2),
        in_specs=[
            pl.BlockSpec((_BLK, _DK), lambda i, kk: (i, kk)),
            pl.BlockSpec((e, _DK), lambda i, kk: (0, 0)),
            pl.BlockSpec((e, _DK), lambda i, kk: (0, 0)),
        ],
        out_specs=[
            pl.BlockSpec((_BLK, e), lambda i, kk: (i, 0)),
            pl.BlockSpec((_BLK, _K), lambda i, kk: (i, 0)),
        ],
        out_shape=[
            jax.ShapeDtypeStruct((b, e), jnp.float32),
            jax.ShapeDtypeStruct((b, _K), jnp.int32),
        ],
        scratch_shapes=[pltpu.VMEM((_BLK, e), jnp.float32)],
        compiler_params=pltpu.CompilerParams(
            dimension_semantics=("parallel", "arbitrary")),
    )(x, w0, w1)
    return fw, idx
